# Initial kernel scaffold; baseline (speedup 1.0000x reference)
import jax
import jax.numpy as jnp
from jax.experimental import pallas as pl


def kernel(centers_a, centers_b):
    raise NotImplementedError("write your pallas kernel here")


# speedup vs baseline: 2145.9721x; 2145.9721x over previous
import jax
import jax.numpy as jnp
from jax import lax
from jax.experimental import pallas as pl
from jax.experimental.pallas import tpu as pltpu

_ALPHA = 0.01
_BG = 2.0
_SWAP = 2.0
_BIG = 1e30
_M = 512
_N = 512


def _soft_red_kernel(a_ref, b_ref, out_ref, troll_ref):
    f32 = jnp.float32
    t = -lax.dot_general(
        b_ref[...], a_ref[...],
        dimension_numbers=(((1,), (1,)), ((), ())),
        preferred_element_type=f32)
    lane2d = lax.broadcasted_iota(jnp.int32, (_M, _N), 1)
    for bit in range(9):
        s = 1 << bit
        rolled = jnp.concatenate([t[_M - s:, :], t[:_M - s, :]], axis=0)
        t = jnp.where((lane2d & s) != 0, rolled, t)
    troll_ref[0:_M, :] = t
    troll_ref[_M:_M + 8, :] = t[0:8, :]

    lvec = lax.broadcasted_iota(jnp.int32, (8, _N), 1)
    big = f32(_BIG)
    zrow = troll_ref[0:8, :] * f32(0.0)
    e_init = zrow + big
    e1_0 = jnp.where(lvec == 0, f32(_BG), big) + zrow

    def rot1(x):
        return pltpu.roll(x, 1, axis=1)

    def rot2(x):
        return pltpu.roll(x, 2, axis=1)

    def load_diag(r):
        base = pl.multiple_of((r >> 3) << 3, 8)
        blk = troll_ref[pl.ds(base, 8), :]
        return pltpu.roll(blk, (8 - (r & 7)) & 7, axis=0)

    def body(k, carry):
        e1, e2, e3, e4, g3 = carry
        kf = k.astype(f32)
        kfb = kf * _BG
        r2 = (k - 2) & 511
        g2 = load_diag(r2)

        sh_e1 = jnp.where(lvec == 0, kfb - _BG, rot1(e1))
        sh_e2 = jnp.where(lvec == 0, kfb - 2.0 * _BG, rot1(e2))
        sh_e4 = jnp.where(lvec == 1, kfb - 4.0 * _BG, rot2(e4))
        sh_g3 = rot1(g3)

        t1 = sh_e2 + g2
        t2 = sh_e1 + _BG
        t3 = e1 + _BG
        t4 = sh_e4 + sh_g3 + g3 + _SWAP
        v4 = (lvec >= 1) & (lvec <= k - 3)
        t4 = jnp.where(v4, t4, big)
        m = jnp.minimum(jnp.minimum(t1, t2), jnp.minimum(t3, t4))
        ssum = (jnp.exp(-(t1 - m) / _ALPHA) + jnp.exp(-(t2 - m) / _ALPHA)
                + jnp.exp(-(t3 - m) / _ALPHA)
                + jnp.where(v4, jnp.exp(-(t4 - m) / _ALPHA), 0.0))
        d = -_ALPHA * jnp.log(ssum) + m
        interior = (lvec <= k - 2) & (lvec >= k - 513)
        e0 = jnp.where(lvec == k - 1, kfb, jnp.where(interior, d, big))
        return (e0, e1, e2, e3, g2)

    g3_0 = zrow
    e_fin = lax.fori_loop(2, _M + _N + 1, body,
                          (e1_0, e_init, e_init, e_init, g3_0))[0]
    svec = lax.broadcasted_iota(jnp.int32, (8, _N), 0)
    sel = jnp.where((svec == 0) & (lvec == _N - 1), e_fin, f32(0.0))
    out_ref[...] = jnp.sum(sel, axis=(0, 1), keepdims=True) * (1.0 / _M)


@jax.jit
def kernel(centers_a, centers_b):
    out = pl.pallas_call(
        _soft_red_kernel,
        out_shape=jax.ShapeDtypeStruct((1, 1), jnp.float32),
        scratch_shapes=[pltpu.VMEM((_M + 8, _N), jnp.float32)],
    )(centers_a, centers_b)
    return out[0, 0]


# speedup vs baseline: 2389.5373x; 1.1135x over previous
import jax
import jax.numpy as jnp
from jax import lax
from jax.experimental import pallas as pl
from jax.experimental.pallas import tpu as pltpu

_ALPHA = 0.01
_BG = 2.0
_SWAP = 2.0
_BIG = 1e30
_M = 512
_N = 512


def _soft_red_kernel(a_ref, b_ref, out_ref, troll_ref):
    f32 = jnp.float32
    t = -lax.dot_general(
        b_ref[...], a_ref[...],
        dimension_numbers=(((1,), (1,)), ((), ())),
        preferred_element_type=f32)
    lane2d = lax.broadcasted_iota(jnp.int32, (_M, _N), 1)
    for bit in range(9):
        s = 1 << bit
        rolled = jnp.concatenate([t[_M - s:, :], t[:_M - s, :]], axis=0)
        t = jnp.where((lane2d & s) != 0, rolled, t)
    troll_ref[0:_M, :] = t
    troll_ref[_M:_M + 8, :] = t[0:8, :]

    lvec = lax.broadcasted_iota(jnp.int32, (8, _N), 1)
    big = f32(_BIG)
    zrow = troll_ref[0:8, :] * f32(0.0)
    e_init = zrow + big
    e1_0 = jnp.where(lvec == 0, f32(_BG), big) + zrow

    def rot1(x):
        return pltpu.roll(x, 1, axis=1)

    def rot2(x):
        return pltpu.roll(x, 2, axis=1)

    def load_diag(r):
        base = pl.multiple_of((r >> 3) << 3, 8)
        blk = troll_ref[pl.ds(base, 8), :]
        return pltpu.roll(blk, (8 - (r & 7)) & 7, axis=0)

    def body(k, carry):
        e1, e2, e3, e4, g3 = carry
        kf = k.astype(f32)
        kfb = kf * _BG
        r2 = (k - 2) & 511
        g2 = load_diag(r2)

        sh_e1 = jnp.where(lvec == 0, kfb - _BG, rot1(e1))
        sh_e2 = jnp.where(lvec == 0, kfb - 2.0 * _BG, rot1(e2))
        sh_e4 = jnp.where(lvec == 1, kfb - 4.0 * _BG, rot2(e4))
        sh_g3 = rot1(g3)

        t1 = sh_e2 + g2
        t2 = sh_e1 + _BG
        t3 = e1 + _BG
        t4 = sh_e4 + sh_g3 + g3 + _SWAP
        v4 = (lvec >= 1) & (lvec <= k - 3)
        t4 = jnp.where(v4, t4, big)
        m = jnp.minimum(jnp.minimum(t1, t2), jnp.minimum(t3, t4))
        ssum = (jnp.exp(-(t1 - m) / _ALPHA) + jnp.exp(-(t2 - m) / _ALPHA)
                + jnp.exp(-(t3 - m) / _ALPHA)
                + jnp.where(v4, jnp.exp(-(t4 - m) / _ALPHA), 0.0))
        d = -_ALPHA * jnp.log(ssum) + m
        interior = (lvec <= k - 2) & (lvec >= k - 513)
        e0 = jnp.where(lvec == k - 1, kfb, jnp.where(interior, d, big))
        return (e0, e1, e2, e3, g2)

    g3_0 = zrow
    e_fin = lax.fori_loop(2, _M + _N + 1, body,
                          (e1_0, e_init, e_init, e_init, g3_0),
                          unroll=4)[0]
    svec = lax.broadcasted_iota(jnp.int32, (8, _N), 0)
    sel = jnp.where((svec == 0) & (lvec == _N - 1), e_fin, f32(0.0))
    out_ref[...] = jnp.sum(sel, axis=(0, 1), keepdims=True) * (1.0 / _M)


@jax.jit
def kernel(centers_a, centers_b):
    out = pl.pallas_call(
        _soft_red_kernel,
        out_shape=jax.ShapeDtypeStruct((1, 1), jnp.float32),
        scratch_shapes=[pltpu.VMEM((_M + 8, _N), jnp.float32)],
    )(centers_a, centers_b)
    return out[0, 0]


# speedup vs baseline: 2527.2218x; 1.0576x over previous
import jax
import jax.numpy as jnp
from jax import lax
from jax.experimental import pallas as pl
from jax.experimental.pallas import tpu as pltpu

_ALPHA = 0.01
_BG = 2.0
_SWAP = 2.0
_BIG = 1e30
_M = 512
_N = 512


def _soft_red_kernel(a_ref, b_ref, out_ref, g_ref, w_ref):
    f32 = jnp.float32
    lane2d = lax.broadcasted_iota(jnp.int32, (_M, 128), 1)
    s_chunks = []
    for c in range(4):
        t_c = -lax.dot_general(
            b_ref[...], a_ref[128 * c:128 * (c + 1), :],
            dimension_numbers=(((1,), (1,)), ((), ())),
            preferred_element_type=f32)
        if c:
            t_c = jnp.concatenate(
                [t_c[_M - 128 * c:, :], t_c[:_M - 128 * c, :]], axis=0)
        for bit in range(7):
            s = 1 << bit
            rolled = jnp.concatenate([t_c[_M - s:, :], t_c[:_M - s, :]],
                                     axis=0)
            t_c = jnp.where((lane2d & s) != 0, rolled, t_c)
        s_chunks.append(t_c)

    rolls = [pltpu.roll(s_c, 1, axis=1) for s_c in s_chunks]
    lane0 = lane2d == 0
    for c in range(4):
        sh_c = jnp.where(lane0, rolls[c - 1] if c else rolls[0], rolls[c])
        w_c = sh_c + s_chunks[c] + _SWAP
        g_ref[c] = s_chunks[c]
        w_ref[c] = w_c
        g_ref[c + 4] = s_chunks[c] * f32(0.0)
        w_ref[c + 4] = s_chunks[c] * f32(0.0)

    snv = lax.broadcasted_iota(jnp.int32, (8, 128), 0)
    lnv = lax.broadcasted_iota(jnp.int32, (8, 128), 1)
    lmat = snv * 128 + lnv
    big = f32(_BIG)
    zrow = jnp.reshape(g_ref[:, 0:1, :], (8, 128)) * f32(0.0)
    e_init = zrow + big
    e1_0 = jnp.where(lmat == 0, f32(_BG), big) + zrow

    def load_row(ref, r):
        return jnp.reshape(ref[:, pl.ds(r, 1), :], (8, 128))

    def shift1(x, inj):
        a = pltpu.roll(x, 1, axis=1)
        a2 = pltpu.roll(a, 1, axis=0)
        r = jnp.where(lnv == 0, a2, a)
        return jnp.where(lmat == 0, inj, r)

    def shift2(x, inj):
        a = pltpu.roll(x, 2, axis=1)
        a2 = pltpu.roll(a, 1, axis=0)
        r = jnp.where(lnv <= 1, a2, a)
        return jnp.where(lmat == 1, inj, r)

    def body(k, carry):
        e1, e2, e3, e4 = carry
        kf = k.astype(f32)
        kfb = kf * _BG
        r2 = (k - 2) & 511
        r3 = (k - 3) & 511
        g2 = load_row(g_ref, r2)
        w3 = load_row(w_ref, r3)

        sh_e1 = shift1(e1, kfb - _BG)
        sh_e2 = shift1(e2, kfb - 2.0 * _BG)
        sh_e4 = shift2(e4, kfb - 4.0 * _BG)

        t1 = sh_e2 + g2
        t2 = sh_e1 + _BG
        t3 = e1 + _BG
        t4 = sh_e4 + w3
        v4 = (lmat >= 1) & (lmat <= k - 3)
        t4 = jnp.where(v4, t4, big)
        m = jnp.minimum(jnp.minimum(t1, t2), jnp.minimum(t3, t4))
        ssum = (jnp.exp(-(t1 - m) / _ALPHA) + jnp.exp(-(t2 - m) / _ALPHA)
                + jnp.exp(-(t3 - m) / _ALPHA)
                + jnp.where(v4, jnp.exp(-(t4 - m) / _ALPHA), 0.0))
        d = -_ALPHA * jnp.log(ssum) + m
        interior = (lmat <= k - 2) & (lmat >= k - 513)
        e0 = jnp.where(lmat == k - 1, kfb, jnp.where(interior, d, big))
        return (e0, e1, e2, e3)

    e_fin = lax.fori_loop(2, _M + _N + 1, body,
                          (e1_0, e_init, e_init, e_init),
                          unroll=4)[0]
    sel = jnp.where(lmat == _N - 1, e_fin, f32(0.0))
    out_ref[...] = jnp.sum(sel, axis=(0, 1), keepdims=True) * (1.0 / _M)


@jax.jit
def kernel(centers_a, centers_b):
    out = pl.pallas_call(
        _soft_red_kernel,
        out_shape=jax.ShapeDtypeStruct((1, 1), jnp.float32),
        scratch_shapes=[pltpu.VMEM((8, _M, 128), jnp.float32),
                        pltpu.VMEM((8, _M, 128), jnp.float32)],
    )(centers_a, centers_b)
    return out[0, 0]
